# Initial kernel scaffold; baseline (speedup 1.0000x reference)
import jax
import jax.numpy as jnp
from jax.experimental import pallas as pl


def kernel(keypoints, boxes, keypoint_logits):
    raise NotImplementedError("write your pallas kernel here")


# speedup vs baseline: 1.0315x; 1.0315x over previous
import functools

import jax
import jax.numpy as jnp
from jax import lax
from jax.experimental import pallas as pl
from jax.experimental.pallas import tpu as pltpu


def _loss_body(params_ref, logits_ref, out_ref, acc_ref, *, rows, length, hm, grid):
    i = pl.program_id(0)

    x = params_ref[:, 0:1]
    y = params_ref[:, 1:2]
    vis = params_ref[:, 2:3]
    ox = params_ref[:, 3:4]
    oy = params_ref[:, 4:5]
    x2 = params_ref[:, 5:6]
    y2 = params_ref[:, 6:7]

    fhm = jnp.float32(hm)
    sx = fhm / (x2 - ox)
    sy = fhm / (y2 - oy)
    xi = jnp.floor((x - ox) * sx).astype(jnp.int32)
    yi = jnp.floor((y - oy) * sy).astype(jnp.int32)
    xi = jnp.where(x == x2, hm - 1, xi)
    yi = jnp.where(y == y2, hm - 1, yi)
    valid = (xi >= 0) & (yi >= 0) & (xi < hm) & (yi < hm) & (vis > 0.0)
    tgt = jnp.where(valid, yi * hm + xi, 0)
    vf = valid.astype(jnp.float32)

    xb = logits_ref[...]
    m = jnp.max(xb, axis=1, keepdims=True)
    s = jnp.sum(jnp.exp(xb - m), axis=1, keepdims=True)
    logz = jnp.log(s) + m

    col = lax.broadcasted_iota(jnp.int32, (rows, length), 1)
    picked = jnp.sum(jnp.where(col == tgt, xb, 0.0), axis=1, keepdims=True)

    part_loss = jnp.sum((logz - picked) * vf)
    part_cnt = jnp.sum(vf)

    @pl.when(i == 0)
    def _init():
        acc_ref[0] = 0.0
        acc_ref[1] = 0.0

    acc_ref[0] += part_loss
    acc_ref[1] += part_cnt

    @pl.when(i == grid - 1)
    def _fin():
        nv = acc_ref[1]
        loss = jnp.where(nv > 0.0, acc_ref[0] / jnp.maximum(nv, 1.0), 0.0)
        out_ref[...] = jnp.reshape(loss, (1, 1))


def kernel(keypoints, boxes, keypoint_logits):
    n, k = keypoint_logits.shape[0], keypoint_logits.shape[1]
    hm = keypoint_logits.shape[-1]
    nr = n * k
    length = keypoint_logits.shape[2] * keypoint_logits.shape[3]

    rows = 512
    grid = nr // rows

    kp = keypoints.reshape(nr, 3)
    bx = jnp.broadcast_to(boxes[:, None, :], (n, k, 4)).reshape(nr, 4)
    pad = jnp.zeros((nr, 1), jnp.float32)
    params = jnp.concatenate([kp, bx, pad], axis=1)

    logits = keypoint_logits.reshape(nr, length)

    body = functools.partial(_loss_body, rows=rows, length=length, hm=hm, grid=grid)
    loss = pl.pallas_call(
        body,
        grid=(grid,),
        in_specs=[
            pl.BlockSpec((rows, 8), lambda i: (i, 0)),
            pl.BlockSpec((rows, length), lambda i: (i, 0)),
        ],
        out_specs=pl.BlockSpec((1, 1), lambda i: (0, 0)),
        out_shape=jax.ShapeDtypeStruct((1, 1), jnp.float32),
        scratch_shapes=[pltpu.SMEM((2,), jnp.float32)],
        compiler_params=pltpu.CompilerParams(
            dimension_semantics=("arbitrary",),
        ),
    )(params, logits)
    return loss[0, 0]
